# baseline (speedup 1.0000x reference)
import functools

import jax
import jax.numpy as jnp
import numpy as np
from jax import lax
from jax.experimental import pallas as pl
from jax.experimental.pallas import tpu as pltpu
from jax.experimental.pallas import tpu_sc as plsc

N = 10000
EMB = 128
HALF = EMB // 2
DEG = 32
EG = N * DEG
NC = 2
NS = 16
LANES = 16
K = 80
EPW = EG // NS
BLK_E = 2000
NBLK = EPW // BLK_E
CPB = BLK_E // K
NZCH = N // K
MAXT = (NZCH + NS - 1) // NS

_PERM = np.empty(EMB, np.int32)
for _h in range(4):
    for _t in range(16):
        _PERM[32 * _h + _t] = 32 * _h + 2 * _t
        _PERM[32 * _h + 16 + _t] = 32 * _h + 2 * _t + 1


@functools.partial(
    pl.kernel,
    out_type=jax.ShapeDtypeStruct((NC, N, EMB), jnp.float32),
    mesh=plsc.VectorSubcoreMesh(core_axis_name="c", subcore_axis_name="s",
                                num_cores=NC, num_subcores=NS),
    compiler_params=pltpu.CompilerParams(use_tc_tiling_on_sc=False),
    scratch_types=[
        pltpu.VMEM_SHARED((N, EMB), jnp.float32),
        pltpu.VMEM((BLK_E,), jnp.int32),
        pltpu.VMEM((BLK_E,), jnp.int32),
        pltpu.VMEM((BLK_E,), jnp.float32),
        pltpu.VMEM((K, HALF), jnp.int32),
        pltpu.VMEM((K, HALF), jnp.int32),
        pltpu.VMEM((LANES, EMB), jnp.float32),
        pltpu.VMEM((K, EMB), jnp.float32),
        pltpu.SemaphoreType.DMA,
        pltpu.SemaphoreType.DMA,
    ],
)
def _spmm_sc(x_hbm, src_hbm, dst_hbm, w_hbm, out_hbm, acc_sh,
             src_v, dst_v, w_v, rows_v, rows_b_v, out_v, zbuf_v, sema, semb):
    c = lax.axis_index("c")
    s = lax.axis_index("s")
    ebase = (c * NS + s) * EPW

    def zrow(r, _):
        for j in range(EMB // LANES):
            zbuf_v[r, pl.ds(j * LANES, LANES)] = jnp.zeros((LANES,),
                                                           jnp.float32)
        return 0
    lax.fori_loop(0, K, zrow, 0)
    for t in range(MAXT):
        idx = s + NS * t
        @pl.when(idx < NZCH)
        def _():
            off = pl.multiple_of(idx * K, 8)
            pltpu.sync_copy(zbuf_v, acc_sh.at[pl.ds(off, K)])
    plsc.subcore_barrier()

    def block(b, _):
        boff = pl.multiple_of(b * BLK_E, 8)
        pltpu.sync_copy(src_hbm.at[pl.ds(ebase + boff, BLK_E)], src_v)
        pltpu.sync_copy(dst_hbm.at[pl.ds(ebase + boff, BLK_E)], dst_v)
        pltpu.sync_copy(w_hbm.at[pl.ds(ebase + boff, BLK_E)], w_v)

        def issue(ci, buf, sem):
            koff = pl.multiple_of(ci * K, 8)
            pltpu.async_copy(x_hbm.at[src_v.at[pl.ds(koff, K)]], buf, sem)

        def drain(buf, sem):
            pltpu.make_async_copy(x_hbm.at[pl.ds(0, K)], buf, sem).wait()

        def proc(ci, buf):
            def group(g, _):
                goff = pl.multiple_of(ci * K + g * LANES, LANES)
                wvec = w_v[pl.ds(goff, LANES)]
                dvec = dst_v[pl.ds(goff, LANES)]
                for el in range(LANES):
                    ws = wvec[el]
                    e = g * LANES + el
                    for h in range(4):
                        pw = buf[e, pl.ds(16 * h, 16)]
                        ve = lax.bitcast_convert_type(
                            pw << 16, jnp.float32) * ws
                        vo = lax.bitcast_convert_type(
                            pw & jnp.int32(-65536), jnp.float32) * ws
                        out_v[el, pl.ds(32 * h, LANES)] = ve
                        out_v[el, pl.ds(32 * h + 16, LANES)] = vo
                pltpu.sync_copy(out_v, acc_sh.at[dvec], add=True)
                return 0
            lax.fori_loop(0, K // LANES, group, 0)

        issue(0, rows_v, sema)
        def pair(h, _):
            ca = 2 * h
            drain(rows_v, sema)
            issue(ca + 1, rows_b_v, semb)
            proc(ca, rows_v)
            drain(rows_b_v, semb)
            issue(ca + 2, rows_v, sema)
            proc(ca + 1, rows_b_v)
            return 0
        lax.fori_loop(0, (CPB - 1) // 2, pair, 0)
        drain(rows_v, sema)
        proc(CPB - 1, rows_v)
        return 0
    lax.fori_loop(0, NBLK, block, 0)

    plsc.subcore_barrier()
    for t in range(MAXT):
        idx = s + NS * t
        @pl.when(idx < NZCH)
        def _():
            off = pl.multiple_of(idx * K, 8)
            pltpu.sync_copy(acc_sh.at[pl.ds(off, K)],
                            out_hbm.at[c, pl.ds(off, K)])


ROWS_BLK = 2000


def _dense_tc_body(p_ref, w_ref, b_ref, o_ref, o16_ref):
    z = jnp.dot(p_ref[0], w_ref[0], preferred_element_type=jnp.float32)
    z = z + b_ref[0, 0:1, :]
    z = jnp.maximum(z, 0.0)
    nrm = jnp.sqrt(jnp.sum(z * z, axis=1, keepdims=True))
    z = z / jnp.maximum(nrm, 1e-12)
    o_ref[0] = z
    o16_ref[0] = z.astype(jnp.bfloat16)


def _dense_tc(p, wstack, bstack):
    bpad = jnp.broadcast_to(bstack[:, None, :], (NC, 8, EMB))
    return pl.pallas_call(
        _dense_tc_body,
        grid=(NC, N // ROWS_BLK),
        in_specs=[
            pl.BlockSpec((1, ROWS_BLK, EMB), lambda g, r: (g, r, 0)),
            pl.BlockSpec((1, EMB, EMB), lambda g, r: (g, 0, 0)),
            pl.BlockSpec((1, 8, EMB), lambda g, r: (g, 0, 0)),
        ],
        out_specs=[
            pl.BlockSpec((1, ROWS_BLK, EMB), lambda g, r: (g, r, 0)),
            pl.BlockSpec((1, ROWS_BLK, EMB), lambda g, r: (g, r, 0)),
        ],
        out_shape=[
            jax.ShapeDtypeStruct((NC, N, EMB), jnp.float32),
            jax.ShapeDtypeStruct((NC, N, EMB), jnp.bfloat16),
        ],
    )(p, wstack, bpad)


def _pack_bf16(xf):
    x16 = xf.astype(jnp.bfloat16).reshape(-1, HALF, 2)
    return lax.bitcast_convert_type(x16, jnp.int32)


def kernel(embedding_user, embedding_item, Wu0, bu0, Wu1, bu1, Wi0, bi0,
           Wi1, bi1, user_edge_weight, item_edge_weight, user_edge_index,
           item_edge_index):
    src = jnp.concatenate([user_edge_index[0], item_edge_index[0] + N])
    dst = jnp.concatenate([user_edge_index[1], item_edge_index[1]])
    w = jnp.concatenate([user_edge_weight, item_edge_weight])

    xi = _pack_bf16(jnp.concatenate([embedding_user, embedding_item], axis=0))
    perm = jnp.asarray(_PERM)
    weights = [(jnp.stack([Wu0, Wi0])[:, perm, :], jnp.stack([bu0, bi0])),
               (jnp.stack([Wu1, Wi1])[:, perm, :], jnp.stack([bu1, bi1]))]
    for l in range(2):
        p = _spmm_sc(xi, src, dst, w)
        y, y16 = _dense_tc(p, weights[l][0], weights[l][1])
        xi = lax.bitcast_convert_type(
            y16.reshape(NC * N, HALF, 2), jnp.int32)
    return (y[0], y[1])


# speedup vs baseline: 2.1608x; 2.1608x over previous
import functools

import jax
import jax.numpy as jnp
import numpy as np
from jax import lax
from jax.experimental import pallas as pl
from jax.experimental.pallas import tpu as pltpu
from jax.experimental.pallas import tpu_sc as plsc

N = 10000
EMB = 128
HALF = EMB // 2
DEG = 32
EG = N * DEG
NC = 2
NS = 16
LANES = 16
K = 80
EPW = EG // NS
BLK_E = 2000
NBLK = EPW // BLK_E
CPB = BLK_E // K
NZCH = N // K
MAXT = (NZCH + NS - 1) // NS

_PERM = np.empty(EMB, np.int32)
for _h in range(4):
    for _t in range(16):
        _PERM[32 * _h + _t] = 32 * _h + 2 * _t
        _PERM[32 * _h + 16 + _t] = 32 * _h + 2 * _t + 1


@functools.partial(
    pl.kernel,
    out_type=jax.ShapeDtypeStruct((NC, N, EMB), jnp.float32),
    mesh=plsc.VectorSubcoreMesh(core_axis_name="c", subcore_axis_name="s",
                                num_cores=NC, num_subcores=NS),
    compiler_params=pltpu.CompilerParams(use_tc_tiling_on_sc=False),
    scratch_types=[
        pltpu.VMEM_SHARED((N, EMB), jnp.float32),
        pltpu.VMEM((BLK_E,), jnp.int32),
        pltpu.VMEM((BLK_E,), jnp.int32),
        pltpu.VMEM((BLK_E,), jnp.float32),
        pltpu.VMEM((K, EMB), jnp.float32),
        pltpu.VMEM((K, EMB), jnp.float32),
        pltpu.VMEM((K, EMB), jnp.float32),
        pltpu.SemaphoreType.DMA,
        pltpu.SemaphoreType.DMA,
    ],
)
def _spmm_sc(x_hbm, src_hbm, dst_hbm, w_hbm, out_hbm, acc_sh,
             src_v, dst_v, w_v, rows_v, rows_b_v, zbuf_v, sema, semb):
    c = lax.axis_index("c")
    s = lax.axis_index("s")
    ebase = (c * NS + s) * EPW

    def zrow(r, _):
        for j in range(EMB // LANES):
            zbuf_v[r, pl.ds(j * LANES, LANES)] = jnp.zeros((LANES,),
                                                           jnp.float32)
        return 0
    lax.fori_loop(0, K, zrow, 0)
    for t in range(MAXT):
        idx = s + NS * t
        @pl.when(idx < NZCH)
        def _():
            off = pl.multiple_of(idx * K, 8)
            pltpu.sync_copy(zbuf_v, acc_sh.at[pl.ds(off, K)])
    plsc.subcore_barrier()

    def block(b, _):
        boff = pl.multiple_of(b * BLK_E, 8)
        pltpu.sync_copy(src_hbm.at[pl.ds(ebase + boff, BLK_E)], src_v)
        pltpu.sync_copy(dst_hbm.at[pl.ds(ebase + boff, BLK_E)], dst_v)
        pltpu.sync_copy(w_hbm.at[pl.ds(ebase + boff, BLK_E)], w_v)

        def issue(ci, buf, sem):
            koff = pl.multiple_of(ci * K, 8)
            pltpu.async_copy(x_hbm.at[src_v.at[pl.ds(koff, K)]], buf, sem)

        def drain(buf, sem):
            pltpu.make_async_copy(x_hbm.at[pl.ds(0, K)], buf, sem).wait()

        def proc(ci, buf):
            def group(g, _):
                goff = pl.multiple_of(ci * K + g * LANES, LANES)
                wvec = w_v[pl.ds(goff, LANES)]
                dvec = dst_v[pl.ds(goff, LANES)]
                for el in range(LANES):
                    ws = wvec[el]
                    e = g * LANES + el
                    for j in range(EMB // LANES):
                        buf[e, pl.ds(j * LANES, LANES)] = (
                            buf[e, pl.ds(j * LANES, LANES)] * ws)
                roff = pl.multiple_of(g * LANES, LANES)
                pltpu.sync_copy(buf.at[pl.ds(roff, LANES)],
                                acc_sh.at[dvec], add=True)
                return 0
            lax.fori_loop(0, K // LANES, group, 0)

        issue(0, rows_v, sema)
        def pair(h, _):
            ca = 2 * h
            drain(rows_v, sema)
            issue(ca + 1, rows_b_v, semb)
            proc(ca, rows_v)
            drain(rows_b_v, semb)
            issue(ca + 2, rows_v, sema)
            proc(ca + 1, rows_b_v)
            return 0
        lax.fori_loop(0, (CPB - 1) // 2, pair, 0)
        drain(rows_v, sema)
        proc(CPB - 1, rows_v)
        return 0
    lax.fori_loop(0, NBLK, block, 0)

    plsc.subcore_barrier()
    for t in range(MAXT):
        idx = s + NS * t
        @pl.when(idx < NZCH)
        def _():
            off = pl.multiple_of(idx * K, 8)
            pltpu.sync_copy(acc_sh.at[pl.ds(off, K)],
                            out_hbm.at[c, pl.ds(off, K)])


ROWS_BLK = 2000


def _dense_tc_body(p_ref, w_ref, b_ref, o_ref, o16_ref):
    z = jnp.dot(p_ref[0], w_ref[0], preferred_element_type=jnp.float32)
    z = z + b_ref[0, 0:1, :]
    z = jnp.maximum(z, 0.0)
    nrm = jnp.sqrt(jnp.sum(z * z, axis=1, keepdims=True))
    z = z / jnp.maximum(nrm, 1e-12)
    o_ref[0] = z
    o16_ref[0] = z.astype(jnp.bfloat16)


def _dense_tc(p, wstack, bstack):
    bpad = jnp.broadcast_to(bstack[:, None, :], (NC, 8, EMB))
    return pl.pallas_call(
        _dense_tc_body,
        grid=(NC, N // ROWS_BLK),
        in_specs=[
            pl.BlockSpec((1, ROWS_BLK, EMB), lambda g, r: (g, r, 0)),
            pl.BlockSpec((1, EMB, EMB), lambda g, r: (g, 0, 0)),
            pl.BlockSpec((1, 8, EMB), lambda g, r: (g, 0, 0)),
        ],
        out_specs=[
            pl.BlockSpec((1, ROWS_BLK, EMB), lambda g, r: (g, r, 0)),
            pl.BlockSpec((1, ROWS_BLK, EMB), lambda g, r: (g, r, 0)),
        ],
        out_shape=[
            jax.ShapeDtypeStruct((NC, N, EMB), jnp.float32),
            jax.ShapeDtypeStruct((NC, N, EMB), jnp.bfloat16),
        ],
    )(p, wstack, bpad)


def _pack_bf16(xf):
    x16 = xf.astype(jnp.bfloat16).reshape(-1, HALF, 2)
    return lax.bitcast_convert_type(x16, jnp.int32)


def kernel(embedding_user, embedding_item, Wu0, bu0, Wu1, bu1, Wi0, bi0,
           Wi1, bi1, user_edge_weight, item_edge_weight, user_edge_index,
           item_edge_index):
    src = jnp.concatenate([user_edge_index[0], item_edge_index[0] + N])
    dst = jnp.concatenate([user_edge_index[1], item_edge_index[1]])
    w = jnp.concatenate([user_edge_weight, item_edge_weight])

    xi = jnp.concatenate([embedding_user, embedding_item], axis=0)
    weights = [(jnp.stack([Wu0, Wi0]), jnp.stack([bu0, bi0])),
               (jnp.stack([Wu1, Wi1]), jnp.stack([bu1, bi1]))]
    for l in range(2):
        p = _spmm_sc(xi, src, dst, w)
        y, y16 = _dense_tc(p, weights[l][0], weights[l][1])
        xi = y.reshape(NC * N, EMB)
    return (y[0], y[1])


# speedup vs baseline: 2.2281x; 1.0311x over previous
import functools

import jax
import jax.numpy as jnp
import numpy as np
from jax import lax
from jax.experimental import pallas as pl
from jax.experimental.pallas import tpu as pltpu
from jax.experimental.pallas import tpu_sc as plsc

N = 10000
EMB = 128
HALF = EMB // 2
DEG = 32
EG = N * DEG
NC = 2
NS = 16
LANES = 16
K = 80
EPW = EG // NS
BLK_E = 2000
NBLK = EPW // BLK_E
CPB = BLK_E // K
NZCH = N // K
MAXT = (NZCH + NS - 1) // NS

_PERM = np.empty(EMB, np.int32)
for _h in range(4):
    for _t in range(16):
        _PERM[32 * _h + _t] = 32 * _h + 2 * _t
        _PERM[32 * _h + 16 + _t] = 32 * _h + 2 * _t + 1


@functools.partial(
    pl.kernel,
    out_type=jax.ShapeDtypeStruct((NC, N, EMB), jnp.float32),
    mesh=plsc.VectorSubcoreMesh(core_axis_name="c", subcore_axis_name="s",
                                num_cores=NC, num_subcores=NS),
    compiler_params=pltpu.CompilerParams(use_tc_tiling_on_sc=False),
    scratch_types=[
        pltpu.VMEM_SHARED((N, EMB), jnp.float32),
        pltpu.VMEM((BLK_E,), jnp.int32),
        pltpu.VMEM((BLK_E,), jnp.int32),
        pltpu.VMEM((BLK_E,), jnp.float32),
        pltpu.VMEM((K, EMB), jnp.float32),
        pltpu.VMEM((K, EMB), jnp.float32),
        pltpu.VMEM((K, EMB), jnp.float32),
        pltpu.SemaphoreType.DMA,
        pltpu.SemaphoreType.DMA,
        pltpu.SemaphoreType.DMA,
        pltpu.SemaphoreType.DMA,
    ],
)
def _spmm_sc(x_hbm, src_hbm, dst_hbm, w_hbm, out_hbm, acc_sh,
             src_v, dst_v, w_v, rows_v, rows_b_v, zbuf_v, sema, sema2,
             semb, semb2):
    c = lax.axis_index("c")
    s = lax.axis_index("s")
    ebase = (c * NS + s) * EPW

    def zrow(r, _):
        for j in range(EMB // LANES):
            zbuf_v[r, pl.ds(j * LANES, LANES)] = jnp.zeros((LANES,),
                                                           jnp.float32)
        return 0
    lax.fori_loop(0, K, zrow, 0)
    for t in range(MAXT):
        idx = s + NS * t
        @pl.when(idx < NZCH)
        def _():
            off = pl.multiple_of(idx * K, 8)
            pltpu.sync_copy(zbuf_v, acc_sh.at[pl.ds(off, K)])
    plsc.subcore_barrier()

    def block(b, _):
        boff = pl.multiple_of(b * BLK_E, 8)
        pltpu.sync_copy(src_hbm.at[pl.ds(ebase + boff, BLK_E)], src_v)
        pltpu.sync_copy(dst_hbm.at[pl.ds(ebase + boff, BLK_E)], dst_v)
        pltpu.sync_copy(w_hbm.at[pl.ds(ebase + boff, BLK_E)], w_v)

        H2 = K // 2

        def issue(ci, buf, sem, sem2):
            koff = pl.multiple_of(ci * K, 8)
            koff2 = pl.multiple_of(ci * K + H2, 8)
            pltpu.async_copy(x_hbm.at[src_v.at[pl.ds(koff, H2)]],
                             buf.at[pl.ds(0, H2)], sem)
            pltpu.async_copy(x_hbm.at[src_v.at[pl.ds(koff2, H2)]],
                             buf.at[pl.ds(H2, H2)], sem2)

        def drain(buf, sem, sem2):
            pltpu.make_async_copy(x_hbm.at[pl.ds(0, H2)],
                                  buf.at[pl.ds(0, H2)], sem).wait()
            pltpu.make_async_copy(x_hbm.at[pl.ds(0, H2)],
                                  buf.at[pl.ds(H2, H2)], sem2).wait()

        def proc(ci, buf):
            def group(g, _):
                goff = pl.multiple_of(ci * K + g * LANES, LANES)
                wvec = w_v[pl.ds(goff, LANES)]
                dvec = dst_v[pl.ds(goff, LANES)]
                for el in range(LANES):
                    ws = wvec[el]
                    e = g * LANES + el
                    for j in range(EMB // LANES):
                        buf[e, pl.ds(j * LANES, LANES)] = (
                            buf[e, pl.ds(j * LANES, LANES)] * ws)
                roff = pl.multiple_of(g * LANES, LANES)
                pltpu.sync_copy(buf.at[pl.ds(roff, LANES)],
                                acc_sh.at[dvec], add=True)
                return 0
            lax.fori_loop(0, K // LANES, group, 0)

        issue(0, rows_v, sema, sema2)
        def pair(h, _):
            ca = 2 * h
            drain(rows_v, sema, sema2)
            issue(ca + 1, rows_b_v, semb, semb2)
            proc(ca, rows_v)
            drain(rows_b_v, semb, semb2)
            issue(ca + 2, rows_v, sema, sema2)
            proc(ca + 1, rows_b_v)
            return 0
        lax.fori_loop(0, (CPB - 1) // 2, pair, 0)
        drain(rows_v, sema, sema2)
        proc(CPB - 1, rows_v)
        return 0
    lax.fori_loop(0, NBLK, block, 0)

    plsc.subcore_barrier()
    for t in range(MAXT):
        idx = s + NS * t
        @pl.when(idx < NZCH)
        def _():
            off = pl.multiple_of(idx * K, 8)
            pltpu.sync_copy(acc_sh.at[pl.ds(off, K)],
                            out_hbm.at[c, pl.ds(off, K)])


ROWS_BLK = 2000


def _dense_tc_body(p_ref, w_ref, b_ref, o_ref, o16_ref):
    z = jnp.dot(p_ref[0], w_ref[0], preferred_element_type=jnp.float32)
    z = z + b_ref[0, 0:1, :]
    z = jnp.maximum(z, 0.0)
    nrm = jnp.sqrt(jnp.sum(z * z, axis=1, keepdims=True))
    z = z / jnp.maximum(nrm, 1e-12)
    o_ref[0] = z
    o16_ref[0] = z.astype(jnp.bfloat16)


def _dense_tc(p, wstack, bstack):
    bpad = jnp.broadcast_to(bstack[:, None, :], (NC, 8, EMB))
    return pl.pallas_call(
        _dense_tc_body,
        grid=(NC, N // ROWS_BLK),
        in_specs=[
            pl.BlockSpec((1, ROWS_BLK, EMB), lambda g, r: (g, r, 0)),
            pl.BlockSpec((1, EMB, EMB), lambda g, r: (g, 0, 0)),
            pl.BlockSpec((1, 8, EMB), lambda g, r: (g, 0, 0)),
        ],
        out_specs=[
            pl.BlockSpec((1, ROWS_BLK, EMB), lambda g, r: (g, r, 0)),
            pl.BlockSpec((1, ROWS_BLK, EMB), lambda g, r: (g, r, 0)),
        ],
        out_shape=[
            jax.ShapeDtypeStruct((NC, N, EMB), jnp.float32),
            jax.ShapeDtypeStruct((NC, N, EMB), jnp.bfloat16),
        ],
    )(p, wstack, bpad)


def _pack_bf16(xf):
    x16 = xf.astype(jnp.bfloat16).reshape(-1, HALF, 2)
    return lax.bitcast_convert_type(x16, jnp.int32)


def kernel(embedding_user, embedding_item, Wu0, bu0, Wu1, bu1, Wi0, bi0,
           Wi1, bi1, user_edge_weight, item_edge_weight, user_edge_index,
           item_edge_index):
    src = jnp.concatenate([user_edge_index[0], item_edge_index[0] + N])
    dst = jnp.concatenate([user_edge_index[1], item_edge_index[1]])
    w = jnp.concatenate([user_edge_weight, item_edge_weight])

    xi = jnp.concatenate([embedding_user, embedding_item], axis=0)
    weights = [(jnp.stack([Wu0, Wi0]), jnp.stack([bu0, bi0])),
               (jnp.stack([Wu1, Wi1]), jnp.stack([bu1, bi1]))]
    for l in range(2):
        p = _spmm_sc(xi, src, dst, w)
        y, y16 = _dense_tc(p, weights[l][0], weights[l][1])
        xi = y.reshape(NC * N, EMB)
    return (y[0], y[1])
